# baseline (speedup 1.0000x reference)
import jax
import jax.numpy as jnp
from jax import lax
from jax.experimental import pallas as pl
from jax.experimental.pallas import tpu as pltpu
from jax.experimental.pallas import tpu_sc as plsc

DIM = 64
SCALE = 8.0
LANES = 16

NUM_CORES = 2
NUM_SUBCORES = 16
NUM_WORKERS = NUM_CORES * NUM_SUBCORES

CHUNK = 512


def _make_body(b_per_w: int, n_chunks: int):
    def body(x_hbm, table_hbm, out_hbm, idx_v, buf0, buf1,
             gsem0, gsem1, wsem0, wsem1):
        wid = lax.axis_index("s") * NUM_CORES + lax.axis_index("c")
        base = wid * b_per_w
        pltpu.sync_copy(x_hbm.at[pl.ds(base, b_per_w)], idx_v)

        bufs = (buf0, buf1)
        gsems = (gsem0, gsem1)
        wsems = (wsem0, wsem1)

        def fire_gather(g):
            b = g % 2
            return pltpu.async_copy(
                table_hbm.at[idx_v.at[pl.ds(g * CHUNK, CHUNK)]],
                bufs[b], gsems[b])

        def fire_write(g):
            b = g % 2
            return pltpu.async_copy(
                bufs[b], out_hbm.at[pl.ds(base + g * CHUNK, CHUNK)],
                wsems[b])

        gd = {0: fire_gather(0)}
        wd = {}
        for g in range(n_chunks):
            b = g % 2
            gd.pop(g).wait()
            buf = bufs[b]

            @pl.loop(0, CHUNK, unroll=8)
            def _scale(r, buf=buf):
                for k in range(DIM // LANES):
                    sl = (r, pl.ds(k * LANES, LANES))
                    buf[sl] = buf[sl] * SCALE

            wd[g] = fire_write(g)
            if g + 1 < n_chunks:
                if g >= 1:
                    wd.pop(g - 1).wait()
                gd[g + 1] = fire_gather(g + 1)
        for g in sorted(wd):
            wd.pop(g).wait()

    return body


def kernel(x, table):
    num_idx = x.size
    b_per_w = num_idx // NUM_WORKERS
    n_chunks = b_per_w // CHUNK
    xf = x.reshape(-1)

    mesh = plsc.VectorSubcoreMesh(core_axis_name="c", subcore_axis_name="s")
    out = pl.kernel(
        _make_body(b_per_w, n_chunks),
        out_type=jax.ShapeDtypeStruct((num_idx, DIM), jnp.float32),
        mesh=mesh,
        compiler_params=pltpu.CompilerParams(use_tc_tiling_on_sc=False),
        scratch_types=[
            pltpu.VMEM((b_per_w,), jnp.int32),
            pltpu.VMEM((CHUNK, DIM), jnp.float32),
            pltpu.VMEM((CHUNK, DIM), jnp.float32),
            pltpu.SemaphoreType.DMA,
            pltpu.SemaphoreType.DMA,
            pltpu.SemaphoreType.DMA,
            pltpu.SemaphoreType.DMA,
        ],
    )(xf, table)
    return out.reshape(*x.shape, DIM)


# speedup vs baseline: 1.0290x; 1.0290x over previous
import jax
import jax.numpy as jnp
from jax import lax
from jax.experimental import pallas as pl
from jax.experimental.pallas import tpu as pltpu
from jax.experimental.pallas import tpu_sc as plsc

DIM = 64
SCALE = 8.0
LANES = 16

NUM_CORES = 2
NUM_SUBCORES = 16
NUM_WORKERS = NUM_CORES * NUM_SUBCORES

CHUNK = 256
NBUF = 4


def _make_body(b_per_w: int, n_chunks: int):
    def body(x_hbm, table_hbm, out_hbm, idx_v, *rest):
        bufs = rest[:NBUF]
        gsems = rest[NBUF:2 * NBUF]
        wsems = rest[2 * NBUF:3 * NBUF]
        wid = lax.axis_index("s") * NUM_CORES + lax.axis_index("c")
        base = wid * b_per_w
        pltpu.sync_copy(x_hbm.at[pl.ds(base, b_per_w)], idx_v)

        def fire_gather(g):
            b = g % NBUF
            return pltpu.async_copy(
                table_hbm.at[idx_v.at[pl.ds(g * CHUNK, CHUNK)]],
                bufs[b], gsems[b])

        def fire_write(g):
            b = g % NBUF
            return pltpu.async_copy(
                bufs[b], out_hbm.at[pl.ds(base + g * CHUNK, CHUNK)],
                wsems[b])

        gd = {}
        wd = {}
        for g in range(min(NBUF - 1, n_chunks)):
            gd[g] = fire_gather(g)
        for g in range(n_chunks):
            b = g % NBUF
            gd.pop(g).wait()
            buf = bufs[b]

            @pl.loop(0, CHUNK, unroll=8)
            def _scale(r, buf=buf):
                for k in range(DIM // LANES):
                    sl = (r, pl.ds(k * LANES, LANES))
                    buf[sl] = buf[sl] * SCALE

            wd[g] = fire_write(g)
            h = g + NBUF - 1
            if h < n_chunks:
                if g >= 1:
                    wd.pop(g - 1).wait()
                gd[h] = fire_gather(h)
        for g in sorted(wd):
            wd.pop(g).wait()

    return body


def kernel(x, table):
    num_idx = x.size
    b_per_w = num_idx // NUM_WORKERS
    n_chunks = b_per_w // CHUNK
    xf = x.reshape(-1)

    mesh = plsc.VectorSubcoreMesh(core_axis_name="c", subcore_axis_name="s")
    out = pl.kernel(
        _make_body(b_per_w, n_chunks),
        out_type=jax.ShapeDtypeStruct((num_idx, DIM), jnp.float32),
        mesh=mesh,
        compiler_params=pltpu.CompilerParams(use_tc_tiling_on_sc=False),
        scratch_types=(
            [pltpu.VMEM((b_per_w,), jnp.int32)]
            + [pltpu.VMEM((CHUNK, DIM), jnp.float32)] * NBUF
            + [pltpu.SemaphoreType.DMA] * (2 * NBUF)
        ),
    )(xf, table)
    return out.reshape(*x.shape, DIM)
